# baseline (speedup 1.0000x reference)
import functools

import jax
import jax.numpy as jnp
from jax import lax
from jax.experimental import pallas as pl
from jax.experimental.pallas import tpu as pltpu
from jax.experimental.pallas import tpu_sc as plsc

B = 16384
L = 200
VOCAB = 1000
D = 64

NC = 2
NS = 16
NW = NC * NS
LANES = 16

ROWS_PER_W = B // NW
GROUPS = ROWS_PER_W // LANES


def _table_kernel(emb_ref, w_ref, b_ref, s_ref):
    prod = emb_ref[...] * w_ref[...][None, :]
    s = jnp.sum(prod, axis=1)
    s_ref[...] = (s + b_ref[0]) * (1.0 / L)


def _pool_body(x_hbm, s_hbm, out_hbm, x_v, s_v, o_v):
    cid = lax.axis_index("c")
    sid = lax.axis_index("s")
    wid = sid * NC + cid
    base = wid * ROWS_PER_W

    pltpu.sync_copy(s_hbm, s_v)
    pltpu.sync_copy(x_hbm.at[pl.ds(base, ROWS_PER_W)], x_v)

    lane = lax.iota(jnp.int32, LANES)
    zero = jnp.zeros((LANES,), jnp.int32)

    def group_body(g, carry):
        row0 = g * LANES
        rows = row0 + lane

        def j_body(j, acc):
            xi = plsc.load_gather(x_v, [rows, zero + j])
            return acc + plsc.load_gather(s_v, [xi])

        acc = lax.fori_loop(0, L, j_body, jnp.zeros((LANES,), jnp.float32),
                            unroll=8)
        res = 1.0 / (1.0 + jnp.exp(-acc))
        plsc.store_scatter(o_v, [rows, zero], res)
        return carry

    lax.fori_loop(0, GROUPS, group_body, 0)
    pltpu.sync_copy(o_v, out_hbm.at[pl.ds(base, ROWS_PER_W)])


def kernel(x, emb, W, b):
    w = W.reshape(D).astype(jnp.float32)
    s_flat = pl.pallas_call(
        _table_kernel,
        out_shape=jax.ShapeDtypeStruct((VOCAB,), jnp.float32),
    )(emb, w, b.astype(jnp.float32))

    mesh = plsc.VectorSubcoreMesh(core_axis_name="c", subcore_axis_name="s")
    pool = functools.partial(
        pl.kernel,
        out_type=jax.ShapeDtypeStruct((B, 1), jnp.float32),
        mesh=mesh,
        scratch_types=[
            pltpu.VMEM((ROWS_PER_W, L), jnp.int32),
            pltpu.VMEM((VOCAB,), jnp.float32),
            pltpu.VMEM((ROWS_PER_W, 1), jnp.float32),
        ],
        compiler_params=pltpu.CompilerParams(
            needs_layout_passes=False, use_tc_tiling_on_sc=False),
    )(_pool_body)
    return pool(x.astype(jnp.int32), s_flat)


# speedup vs baseline: 2.5077x; 2.5077x over previous
import functools

import jax
import jax.numpy as jnp
from jax import lax
from jax.experimental import pallas as pl
from jax.experimental.pallas import tpu as pltpu
from jax.experimental.pallas import tpu_sc as plsc

B = 16384
L = 200
VOCAB = 1000
D = 64

NC = 2
NS = 16
NW = NC * NS
LANES = 16

ROWS_PER_W = B // NW
GROUPS = ROWS_PER_W // LANES


def _table_kernel(emb_t_ref, w_ref, b_ref, s_ref):
    prod = emb_t_ref[...] * w_ref[...][:, None]
    s = jnp.sum(prod, axis=0)
    s_ref[...] = (s + b_ref[0]) * (1.0 / L)


def _pool_body(xt_hbm, s_hbm, out_hbm, x_v, s_v, o_v):
    cid = lax.axis_index("c")
    sid = lax.axis_index("s")
    wid = sid * NC + cid
    base = wid * ROWS_PER_W

    pltpu.sync_copy(s_hbm, s_v)
    pltpu.sync_copy(xt_hbm.at[:, pl.ds(base, ROWS_PER_W)], x_v)

    def group_body(g, carry):
        i0 = g * LANES

        def j_body(j, acc):
            xi = x_v[j, pl.ds(i0, LANES)]
            return acc + plsc.load_gather(s_v, [xi])

        acc = lax.fori_loop(0, L, j_body, jnp.zeros((LANES,), jnp.float32),
                            unroll=8)
        res = 1.0 / (1.0 + jnp.exp(-acc))
        o_v[pl.ds(i0, LANES)] = res
        return carry

    lax.fori_loop(0, GROUPS, group_body, 0)
    pltpu.sync_copy(o_v, out_hbm.at[pl.ds(base, ROWS_PER_W)])


def kernel(x, emb, W, b):
    w = W.reshape(D).astype(jnp.float32)
    s_flat = pl.pallas_call(
        _table_kernel,
        out_shape=jax.ShapeDtypeStruct((VOCAB,), jnp.float32),
    )(emb.T, w, b.astype(jnp.float32))

    mesh = plsc.VectorSubcoreMesh(core_axis_name="c", subcore_axis_name="s")
    pool = functools.partial(
        pl.kernel,
        out_type=jax.ShapeDtypeStruct((B,), jnp.float32),
        mesh=mesh,
        scratch_types=[
            pltpu.VMEM((L, ROWS_PER_W), jnp.int32),
            pltpu.VMEM((VOCAB,), jnp.float32),
            pltpu.VMEM((ROWS_PER_W,), jnp.float32),
        ],
        compiler_params=pltpu.CompilerParams(needs_layout_passes=False),
    )(_pool_body)
    out = pool(x.T.astype(jnp.int32), s_flat)
    return out.reshape(B, 1)
